# baseline (speedup 1.0000x reference)
import jax
import jax.numpy as jnp
from jax import lax
from jax.experimental import pallas as pl
from jax.experimental.pallas import tpu as pltpu
from jax.experimental.pallas import tpu_sc as plsc

N_NODES_C = 10000
N_EDGES_C = 320000
D_C = 128

NC = 2
NS = 16
NW = NC * NS

E_PER_W = N_EDGES_C // NW
CHA = 80
NCHA = E_PER_W // CHA
CHB = 40
NCHB = E_PER_W // CHB
ZCHUNK = 640


def _tile_rows(s):
    return pl.multiple_of(s * ZCHUNK - (s // (NS - 1)) * 240, 8)


def _body_pool(edges_hbm, ei0_hbm, z_hbm, part_hbm, pool, eb, idx_v,
               sem_eg, sem_sa):
    c = lax.axis_index("c")
    s = lax.axis_index("s")
    w = s * NC + c

    zoff = _tile_rows(s)

    def zero(kz, _):
        r0 = pl.multiple_of(zoff + kz * CHA, 8)
        pltpu.sync_copy(z_hbm.at[pl.ds(r0, CHA)], pool.at[pl.ds(r0, CHA)])
        return 0

    lax.fori_loop(0, ZCHUNK // CHA, zero, 0)
    plsc.subcore_barrier()

    pltpu.sync_copy(ei0_hbm.at[w], idx_v)
    base = w * E_PER_W

    def start_eg(k, b):
        pltpu.async_copy(edges_hbm.at[pl.ds(base + k * CHA, CHA)],
                         eb.at[b], sem_eg.at[b])

    def wait_eg(b):
        pltpu.make_async_copy(edges_hbm.at[pl.ds(0, CHA)], eb.at[b],
                              sem_eg.at[b]).wait()

    def start_sa(k, b):
        pltpu.async_copy(eb.at[b], pool.at[idx_v.at[k]], sem_sa.at[b],
                         add=True)

    def wait_sa(b):
        pltpu.make_async_copy(eb.at[b], pool.at[pl.ds(0, CHA)],
                              sem_sa.at[b]).wait()

    start_eg(0, 0)
    start_eg(1, 1)
    start_eg(2, 2)
    wait_eg(0)
    start_sa(0, 0)
    wait_sa(0)
    start_eg(3, 0)
    wait_eg(1)
    start_sa(1, 1)

    def body(k, _):
        b = lax.rem(k, 3)
        bf = lax.rem(k + 2, 3)
        wait_sa(bf)
        start_eg(k + 2, bf)
        wait_eg(b)
        start_sa(k, b)
        return 0

    lax.fori_loop(2, NCHA - 2, body, 0)

    for k in (NCHA - 2, NCHA - 1):
        wait_eg(k % 3)
        start_sa(k, k % 3)
    for k in range(NCHA - 3, NCHA):
        wait_sa(k % 3)

    plsc.subcore_barrier()

    def wr(kz, _):
        r0 = pl.multiple_of(zoff + kz * CHA, 8)
        pltpu.sync_copy(pool.at[pl.ds(r0, CHA)],
                        part_hbm.at[c].at[pl.ds(r0, CHA)])
        return 0

    lax.fori_loop(0, ZCHUNK // CHA, wr, 0)


def _body_out(part_hbm, edges_hbm, ei1_hbm, ep_hbm, out_hbm,
              pool, pb, eb, idx1_v, idxp_v, sem_gp, sem_ge, sem_so):
    c = lax.axis_index("c")
    s = lax.axis_index("s")
    w = s * NC + c

    zoff = _tile_rows(s)

    def comb(k, _):
        r0 = pl.multiple_of(zoff + k * CHB, 8)
        pltpu.sync_copy(part_hbm.at[0].at[pl.ds(r0, CHB)], pb.at[0])
        pltpu.sync_copy(part_hbm.at[1].at[pl.ds(r0, CHB)], pb.at[1])

        @plsc.parallel_loop(0, CHB, unroll=2)
        def add_row(r):
            for cc in range(D_C // 16):
                sl = pl.ds(cc * 16, 16)
                pb[1, r, sl] = pb[1, r, sl] + pb[0, r, sl]

        pltpu.sync_copy(pb.at[1], pool.at[pl.ds(r0, CHB)])
        return 0

    lax.fori_loop(0, ZCHUNK // CHB, comb, 0)
    plsc.subcore_barrier()

    pltpu.sync_copy(ei1_hbm.at[w, 0], idx1_v)
    pltpu.sync_copy(ep_hbm.at[w, 0], idxp_v)
    base = w * E_PER_W

    def start_gp(k, b):
        pltpu.async_copy(pool.at[idx1_v.at[pl.ds(k * CHB, CHB)]],
                         pb.at[b], sem_gp.at[b])

    def wait_gp(b):
        pltpu.make_async_copy(edges_hbm.at[pl.ds(0, CHB)], pb.at[b],
                              sem_gp.at[b]).wait()

    def start_ge(k, b):
        pltpu.async_copy(edges_hbm.at[idxp_v.at[pl.ds(k * CHB, CHB)]],
                         eb.at[b], sem_ge.at[b])

    def wait_ge(b):
        pltpu.make_async_copy(edges_hbm.at[pl.ds(0, CHB)], eb.at[b],
                              sem_ge.at[b]).wait()

    def start_so(k, b):
        pltpu.async_copy(pb.at[b], out_hbm.at[pl.ds(base + k * CHB, CHB)],
                         sem_so.at[b])

    def wait_so(b):
        pltpu.make_async_copy(pb.at[b], out_hbm.at[pl.ds(0, CHB)],
                              sem_so.at[b]).wait()

    def subtract(bp, be):
        @plsc.parallel_loop(0, CHB, unroll=8)
        def sub_row(r):
            for cc in range(D_C // 16):
                sl = pl.ds(cc * 16, 16)
                pb[bp, r, sl] = pb[bp, r, sl] - eb[be, r, sl]

    start_gp(0, 0)
    start_gp(1, 1)
    start_ge(0, 0)
    start_gp(2, 2)
    start_ge(1, 1)
    wait_gp(0)
    wait_ge(0)
    subtract(0, 0)
    start_so(0, 0)
    start_gp(3, 3)
    start_ge(2, 0)
    wait_gp(1)
    wait_ge(1)
    subtract(1, 1)
    start_so(1, 1)

    def body(k, _):
        bp = lax.rem(k, 4)
        bpf = lax.rem(k + 2, 4)
        be = lax.rem(k, 2)
        bef = lax.rem(k + 1, 2)
        wait_so(bpf)
        start_gp(k + 2, bpf)
        start_ge(k + 1, bef)
        wait_gp(bp)
        wait_ge(be)
        subtract(bp, be)
        start_so(k, bp)
        return 0

    lax.fori_loop(2, NCHB - 2, body, 0)

    start_ge(NCHB - 1, (NCHB - 1) % 2)
    for k in (NCHB - 2, NCHB - 1):
        wait_gp(k % 4)
        wait_ge(k % 2)
        subtract(k % 4, k % 2)
        start_so(k, k % 4)
    for k in range(NCHB - 4, NCHB):
        wait_so(k % 4)


@jax.jit
def _run(edges, ei0, ei1, ep, z):
    mesh = plsc.VectorSubcoreMesh(core_axis_name="c", subcore_axis_name="s")
    part = pl.kernel(
        _body_pool,
        out_type=jax.ShapeDtypeStruct((NC, N_NODES_C, D_C), jnp.float32),
        mesh=mesh,
        scratch_types=[
            pltpu.VMEM_SHARED((N_NODES_C, D_C), jnp.float32),
            pltpu.VMEM((3, CHA, D_C), jnp.float32),
            pltpu.VMEM((NCHA, CHA), jnp.int32),
            pltpu.SemaphoreType.DMA((3,)),
            pltpu.SemaphoreType.DMA((3,)),
        ],
    )(edges, ei0, z)
    out = pl.kernel(
        _body_out,
        out_type=jax.ShapeDtypeStruct((N_EDGES_C, D_C), jnp.float32),
        mesh=mesh,
        scratch_types=[
            pltpu.VMEM_SHARED((N_NODES_C, D_C), jnp.float32),
            pltpu.VMEM((4, CHB, D_C), jnp.float32),
            pltpu.VMEM((2, CHB, D_C), jnp.float32),
            pltpu.VMEM((E_PER_W,), jnp.int32),
            pltpu.VMEM((E_PER_W,), jnp.int32),
            pltpu.SemaphoreType.DMA((4,)),
            pltpu.SemaphoreType.DMA((2,)),
            pltpu.SemaphoreType.DMA((4,)),
        ],
    )(part, edges, ei1, ep)
    return out


def kernel(nodes, edges, edge_index, edge_pair):
    ei0 = edge_index[0].astype(jnp.int32).reshape(NW, NCHA, CHA)
    ei1 = edge_index[1].astype(jnp.int32).reshape(NW, 1, E_PER_W)
    ep = edge_pair[0].astype(jnp.int32).reshape(NW, 1, E_PER_W)
    z = jnp.zeros((N_NODES_C, D_C), jnp.float32)
    return _run(edges, ei0, ei1, ep, z)


# speedup vs baseline: 1.0616x; 1.0616x over previous
import jax
import jax.numpy as jnp
from jax import lax
from jax.experimental import pallas as pl
from jax.experimental.pallas import tpu as pltpu
from jax.experimental.pallas import tpu_sc as plsc

N_NODES_C = 10000
N_EDGES_C = 320000
D_C = 128

NC = 2
NS = 16
NW = NC * NS

E_PER_W = N_EDGES_C // NW
CHA = 80
NCHA = E_PER_W // CHA
CHB = 40
NCHB = E_PER_W // CHB
ZCHUNK = 640


def _tile_rows(s):
    return pl.multiple_of(s * ZCHUNK - (s // (NS - 1)) * 240, 8)


def _body_pool(edges_hbm, ei0_hbm, z_hbm, part_hbm, pool, eb, idx_v,
               sem_eg, sem_sa):
    c = lax.axis_index("c")
    s = lax.axis_index("s")
    w = s * NC + c

    zoff = _tile_rows(s)

    def zero(kz, _):
        r0 = pl.multiple_of(zoff + kz * CHA, 8)
        pltpu.sync_copy(z_hbm.at[pl.ds(r0, CHA)], pool.at[pl.ds(r0, CHA)])
        return 0

    lax.fori_loop(0, ZCHUNK // CHA, zero, 0)
    plsc.subcore_barrier()

    pltpu.sync_copy(ei0_hbm.at[w], idx_v)
    base = w * E_PER_W

    def start_eg(k, b):
        pltpu.async_copy(edges_hbm.at[pl.ds(base + k * CHA, CHA)],
                         eb.at[b], sem_eg.at[b])

    def wait_eg(b):
        pltpu.make_async_copy(edges_hbm.at[pl.ds(0, CHA)], eb.at[b],
                              sem_eg.at[b]).wait()

    def start_sa(k, b):
        pltpu.async_copy(eb.at[b], pool.at[idx_v.at[k]], sem_sa.at[b],
                         add=True)

    def wait_sa(b):
        pltpu.make_async_copy(eb.at[b], pool.at[pl.ds(0, CHA)],
                              sem_sa.at[b]).wait()

    start_eg(0, 0)
    start_eg(1, 1)
    start_eg(2, 2)
    wait_eg(0)
    start_sa(0, 0)
    wait_sa(0)
    start_eg(3, 0)
    wait_eg(1)
    start_sa(1, 1)

    def body(k, _):
        b = lax.rem(k, 3)
        bf = lax.rem(k + 2, 3)
        wait_sa(bf)
        start_eg(k + 2, bf)
        wait_eg(b)
        start_sa(k, b)
        return 0

    lax.fori_loop(2, NCHA - 2, body, 0)

    for k in (NCHA - 2, NCHA - 1):
        wait_eg(k % 3)
        start_sa(k, k % 3)
    for k in range(NCHA - 3, NCHA):
        wait_sa(k % 3)

    plsc.subcore_barrier()

    def wr(kz, _):
        r0 = pl.multiple_of(zoff + kz * CHA, 8)
        pltpu.sync_copy(pool.at[pl.ds(r0, CHA)],
                        part_hbm.at[c].at[pl.ds(r0, CHA)])
        return 0

    lax.fori_loop(0, ZCHUNK // CHA, wr, 0)


def _body_out(part_hbm, edges_hbm, ei1_hbm, ep_hbm, out_hbm,
              pool, pb, eb, idx1_v, idxp_v, sem_gp, sem_ge, sem_so):
    c = lax.axis_index("c")
    s = lax.axis_index("s")
    w = s * NC + c

    zoff = _tile_rows(s)

    NZB = ZCHUNK // CHB

    def load_parts(k, p):
        r0 = pl.multiple_of(zoff + k * CHB, 8)
        pltpu.async_copy(part_hbm.at[0].at[pl.ds(r0, CHB)], pb.at[2 * p],
                         sem_gp.at[2 * p])
        pltpu.async_copy(part_hbm.at[1].at[pl.ds(r0, CHB)], pb.at[2 * p + 1],
                         sem_gp.at[2 * p + 1])

    def comb_one(k, p):
        r0 = pl.multiple_of(zoff + k * CHB, 8)
        for i in (2 * p, 2 * p + 1):
            pltpu.make_async_copy(part_hbm.at[0].at[pl.ds(0, CHB)],
                                  pb.at[i], sem_gp.at[i]).wait()

        @plsc.parallel_loop(0, CHB, unroll=2)
        def add_row(r):
            for cc in range(D_C // 16):
                sl = pl.ds(cc * 16, 16)
                pb[2 * p + 1, r, sl] = pb[2 * p + 1, r, sl] + pb[2 * p, r, sl]

        pltpu.sync_copy(pb.at[2 * p + 1], pool.at[pl.ds(r0, CHB)])

    load_parts(0, 0)

    def comb(k2, _):
        for p in (0, 1):
            k = 2 * k2 + p
            load_parts(k + 1, 1 - p)
            comb_one(k, p)
        return 0

    lax.fori_loop(0, NZB // 2 - 1, comb, 0)
    for p in (0, 1):
        k = NZB - 2 + p
        if k + 1 < NZB:
            load_parts(k + 1, 1 - p)
        comb_one(k, p)
    plsc.subcore_barrier()

    pltpu.sync_copy(ei1_hbm.at[w, 0], idx1_v)
    pltpu.sync_copy(ep_hbm.at[w, 0], idxp_v)
    base = w * E_PER_W

    def start_gp(k, b):
        pltpu.async_copy(pool.at[idx1_v.at[pl.ds(k * CHB, CHB)]],
                         pb.at[b], sem_gp.at[b])

    def wait_gp(b):
        pltpu.make_async_copy(edges_hbm.at[pl.ds(0, CHB)], pb.at[b],
                              sem_gp.at[b]).wait()

    def start_ge(k, b):
        pltpu.async_copy(edges_hbm.at[idxp_v.at[pl.ds(k * CHB, CHB)]],
                         eb.at[b], sem_ge.at[b])

    def wait_ge(b):
        pltpu.make_async_copy(edges_hbm.at[pl.ds(0, CHB)], eb.at[b],
                              sem_ge.at[b]).wait()

    def start_so(k, b):
        pltpu.async_copy(pb.at[b], out_hbm.at[pl.ds(base + k * CHB, CHB)],
                         sem_so.at[b])

    def wait_so(b):
        pltpu.make_async_copy(pb.at[b], out_hbm.at[pl.ds(0, CHB)],
                              sem_so.at[b]).wait()

    def subtract(bp, be):
        @plsc.parallel_loop(0, CHB, unroll=8)
        def sub_row(r):
            for cc in range(D_C // 16):
                sl = pl.ds(cc * 16, 16)
                pb[bp, r, sl] = pb[bp, r, sl] - eb[be, r, sl]

    start_gp(0, 0)
    start_gp(1, 1)
    start_ge(0, 0)
    start_gp(2, 2)
    start_ge(1, 1)
    wait_gp(0)
    wait_ge(0)
    subtract(0, 0)
    start_so(0, 0)
    start_gp(3, 3)
    start_ge(2, 0)
    wait_gp(1)
    wait_ge(1)
    subtract(1, 1)
    start_so(1, 1)

    def body(k, _):
        bp = lax.rem(k, 4)
        bpf = lax.rem(k + 2, 4)
        be = lax.rem(k, 2)
        bef = lax.rem(k + 1, 2)
        wait_so(bpf)
        start_gp(k + 2, bpf)
        start_ge(k + 1, bef)
        wait_gp(bp)
        wait_ge(be)
        subtract(bp, be)
        start_so(k, bp)
        return 0

    lax.fori_loop(2, NCHB - 2, body, 0)

    start_ge(NCHB - 1, (NCHB - 1) % 2)
    for k in (NCHB - 2, NCHB - 1):
        wait_gp(k % 4)
        wait_ge(k % 2)
        subtract(k % 4, k % 2)
        start_so(k, k % 4)
    for k in range(NCHB - 4, NCHB):
        wait_so(k % 4)


@jax.jit
def _run(edges, ei0, ei1, ep, z):
    mesh = plsc.VectorSubcoreMesh(core_axis_name="c", subcore_axis_name="s")
    part = pl.kernel(
        _body_pool,
        out_type=jax.ShapeDtypeStruct((NC, N_NODES_C, D_C), jnp.float32),
        mesh=mesh,
        scratch_types=[
            pltpu.VMEM_SHARED((N_NODES_C, D_C), jnp.float32),
            pltpu.VMEM((3, CHA, D_C), jnp.float32),
            pltpu.VMEM((NCHA, CHA), jnp.int32),
            pltpu.SemaphoreType.DMA((3,)),
            pltpu.SemaphoreType.DMA((3,)),
        ],
    )(edges, ei0, z)
    out = pl.kernel(
        _body_out,
        out_type=jax.ShapeDtypeStruct((N_EDGES_C, D_C), jnp.float32),
        mesh=mesh,
        scratch_types=[
            pltpu.VMEM_SHARED((N_NODES_C, D_C), jnp.float32),
            pltpu.VMEM((4, CHB, D_C), jnp.float32),
            pltpu.VMEM((2, CHB, D_C), jnp.float32),
            pltpu.VMEM((E_PER_W,), jnp.int32),
            pltpu.VMEM((E_PER_W,), jnp.int32),
            pltpu.SemaphoreType.DMA((4,)),
            pltpu.SemaphoreType.DMA((2,)),
            pltpu.SemaphoreType.DMA((4,)),
        ],
    )(part, edges, ei1, ep)
    return out


def kernel(nodes, edges, edge_index, edge_pair):
    ei0 = edge_index[0].astype(jnp.int32).reshape(NW, NCHA, CHA)
    ei1 = edge_index[1].astype(jnp.int32).reshape(NW, 1, E_PER_W)
    ep = edge_pair[0].astype(jnp.int32).reshape(NW, 1, E_PER_W)
    z = jnp.zeros((N_NODES_C, D_C), jnp.float32)
    return _run(edges, ei0, ei1, ep, z)
